# baseline (speedup 1.0000x reference)
import functools

import jax
import jax.numpy as jnp
from jax import lax
from jax.experimental import pallas as pl
from jax.experimental.pallas import tpu as pltpu
from jax.experimental.pallas import tpu_sc as plsc

N_NODES = 10000
N_EDGES = 320000
IN_DIM = 128
HID_DIM = 64
OUT_DIM = 128

NC = 2
NS = 16
NW = NC * NS
EPW = N_EDGES // NW
B = 128
NB = 80
PADE = NB * B - EPW
NBUF = 4
NPAD = 10240
RPS = NPAD // NS
DEG_W = 8

@functools.lru_cache(maxsize=None)
def _mesh():
    return plsc.VectorSubcoreMesh(
        core_axis_name="c", subcore_axis_name="s", num_cores=NC, num_subcores=NS
    )


def _deg_body(dst_hbm, ones_hbm, zeros_hbm, out_hbm,
              didx, ones_v, ssem, acc_sh, psem):
    c = lax.axis_index("c")
    s = lax.axis_index("s")
    wid = c * NS + s
    zdesc = pltpu.make_async_copy(
        zeros_hbm.at[pl.ds(s * RPS, RPS)], acc_sh.at[pl.ds(s * RPS, RPS)], psem
    )
    zdesc.start()
    pltpu.sync_copy(dst_hbm.at[wid], didx)
    pltpu.sync_copy(ones_hbm, ones_v)
    zdesc.wait()
    plsc.subcore_barrier()

    def scat(b, k):
        return pltpu.make_async_copy(ones_v, acc_sh.at[didx.at[b]], ssem[k])

    for k in range(NBUF):
        scat(k, k).start(add=True)

    def body(g, carry):
        for k in range(NBUF):
            b = g * NBUF + k
            scat(b, k).wait()
            scat(b + NBUF, k).start(add=True)
        return carry

    lax.fori_loop(0, NB // NBUF - 1, body, 0)
    for k in range(NBUF):
        scat(NB - NBUF + k, k).wait()
    plsc.subcore_barrier()
    pltpu.sync_copy(
        acc_sh.at[pl.ds(s * RPS, RPS)], out_hbm.at[c, pl.ds(s * RPS, RPS)]
    )


@functools.lru_cache(maxsize=None)
def _deg_kernel():
    return pl.kernel(
        _deg_body,
        out_type=jax.ShapeDtypeStruct((NC, NPAD, DEG_W), jnp.float32),
        mesh=_mesh(),
        scratch_types=[
            pltpu.VMEM((NB, B), jnp.int32),
            pltpu.VMEM((B, DEG_W), jnp.float32),
            tuple(pltpu.SemaphoreType.DMA for _ in range(NBUF)),
            pltpu.VMEM_SHARED((NPAD, DEG_W), jnp.float32),
            pltpu.SemaphoreType.DMA,
        ],
        compiler_params=pltpu.CompilerParams(use_tc_tiling_on_sc=False),
    )


def _make_agg(feat):
    def _agg_body(h_hbm, src_hbm, dst_hbm, zeros_hbm, out_hbm,
                  sidx, didx, rows, gsem, ssem, acc_sh, psem):
        c = lax.axis_index("c")
        s = lax.axis_index("s")
        wid = c * NS + s
        zdesc = pltpu.make_async_copy(
            zeros_hbm.at[pl.ds(s * RPS, RPS)],
            acc_sh.at[pl.ds(s * RPS, RPS)], psem
        )
        zdesc.start()
        pltpu.sync_copy(src_hbm.at[wid], sidx)
        pltpu.sync_copy(dst_hbm.at[wid], didx)
        zdesc.wait()
        plsc.subcore_barrier()

        def gat(b, k):
            return pltpu.make_async_copy(h_hbm.at[sidx.at[b]], rows[k], gsem[k])

        def scat(b, k):
            return pltpu.make_async_copy(rows[k], acc_sh.at[didx.at[b]], ssem[k])

        for k in range(NBUF):
            gat(k, k).start()

        def body(g, carry):
            for k in range(NBUF):
                b = g * NBUF + k
                gat(b, k).wait()
                scat(b, k).start(add=True)
            for k in range(NBUF):
                b = g * NBUF + k
                scat(b, k).wait()

                @pl.when(b + NBUF < NB)
                def _():
                    gat(b + NBUF, k).start()
            return carry

        lax.fori_loop(0, NB // NBUF, body, 0)
        plsc.subcore_barrier()
        pltpu.sync_copy(
            acc_sh.at[pl.ds(s * RPS, RPS)],
            out_hbm.at[c, pl.ds(s * RPS, RPS), pl.ds(0, feat)],
        )

    return pl.kernel(
        _agg_body,
        out_type=jax.ShapeDtypeStruct((NC, NPAD, 2 * feat), jnp.float32),
        mesh=_mesh(),
        scratch_types=[
            pltpu.VMEM((NB, B), jnp.int32),
            pltpu.VMEM((NB, B), jnp.int32),
            tuple(pltpu.VMEM((B, feat), jnp.float32) for _ in range(NBUF)),
            tuple(pltpu.SemaphoreType.DMA for _ in range(NBUF)),
            tuple(pltpu.SemaphoreType.DMA for _ in range(NBUF)),
            pltpu.VMEM_SHARED((NPAD, feat), jnp.float32),
            pltpu.SemaphoreType.DMA,
        ],
        compiler_params=pltpu.CompilerParams(use_tc_tiling_on_sc=False),
    )


_make_agg = functools.lru_cache(maxsize=None)(_make_agg)


def _mm_scale_body(x_ref, w_ref, dp_ref, o_ref, d_ref):
    dp = dp_ref[...]
    deg = dp[0, :N_NODES, :1] + dp[1, :N_NODES, :1] + 1.0
    dinv = lax.rsqrt(deg)
    h = jnp.dot(x_ref[...], w_ref[...], preferred_element_type=jnp.float32)
    h1s = h * dinv
    o_ref[...] = jnp.concatenate([h1s, jnp.zeros_like(h1s)], axis=1)
    d_ref[...] = jnp.broadcast_to(dinv, (N_NODES, OUT_DIM))


def _mid_body(agg_ref, hs_ref, dv_ref, b_ref, g_ref, bt_ref, w_ref, o_ref):
    dinv = dv_ref[:, :1]
    agg = agg_ref[...]
    acc = (agg[0, :N_NODES, :HID_DIM] + agg[1, :N_NODES, :HID_DIM]
           + hs_ref[:, :HID_DIM])
    pre = acc * dinv + b_ref[...]
    mu = jnp.mean(pre, axis=0, keepdims=True)
    var = jnp.mean((pre - mu) ** 2, axis=0, keepdims=True)
    bn = (pre - mu) * lax.rsqrt(var + 1e-5) * g_ref[...] + bt_ref[...]
    r = jnp.maximum(bn, 0.0)
    h2 = jnp.dot(r, w_ref[...], preferred_element_type=jnp.float32)
    o_ref[...] = h2 * dinv


def _final_body(agga_ref, aggb_ref, hs_ref, dv_ref,
                b_ref, g_ref, bt_ref, o_ref):
    dinv = dv_ref[:, :1]
    acc = jnp.concatenate(
        [agga_ref[0, :N_NODES, :HID_DIM] + agga_ref[1, :N_NODES, :HID_DIM],
         aggb_ref[0, :N_NODES, :HID_DIM] + aggb_ref[1, :N_NODES, :HID_DIM]],
        axis=1,
    ) + hs_ref[...]
    pre = acc * dinv + b_ref[...]
    mu = jnp.mean(pre, axis=0, keepdims=True)
    var = jnp.mean((pre - mu) ** 2, axis=0, keepdims=True)
    o_ref[...] = (pre - mu) * lax.rsqrt(var + 1e-5) * g_ref[...] + bt_ref[...]


def kernel(x, edge_index, W1, b1, g1, bt1, W2, b2, g2, bt2):
    dst = jnp.concatenate(
        [edge_index[1].reshape(NW, EPW),
         jnp.full((NW, PADE), NPAD - 1, jnp.int32)], axis=1
    ).reshape(NW, NB, B)
    src_a = jnp.concatenate(
        [(2 * edge_index[0]).reshape(NW, EPW),
         jnp.zeros((NW, PADE), jnp.int32)], axis=1
    ).reshape(NW, NB, B)
    src_b = src_a + 1
    ones8 = jnp.ones((B, DEG_W), jnp.float32)
    zeros8 = jnp.zeros((NPAD, DEG_W), jnp.float32)
    zeros64 = jnp.zeros((NPAD, HID_DIM), jnp.float32)

    deg_parts = _deg_kernel()(dst, ones8, zeros8)

    h1s, dinv_b = pl.pallas_call(
        _mm_scale_body,
        out_shape=(jax.ShapeDtypeStruct((N_NODES, OUT_DIM), jnp.float32),
                   jax.ShapeDtypeStruct((N_NODES, OUT_DIM), jnp.float32)),
    )(x, W1, deg_parts)

    agg = _make_agg(HID_DIM)
    agg1 = agg(h1s.reshape(2 * N_NODES, HID_DIM), src_a, dst, zeros64)

    h2s = pl.pallas_call(
        _mid_body,
        out_shape=jax.ShapeDtypeStruct((N_NODES, OUT_DIM), jnp.float32),
    )(agg1, h1s, dinv_b, b1.reshape(1, -1), g1.reshape(1, -1),
      bt1.reshape(1, -1), W2)

    h2v = h2s.reshape(2 * N_NODES, HID_DIM)
    agg2a = agg(h2v, src_a, dst, zeros64)
    agg2b = agg(h2v, src_b, dst, zeros64)

    out = pl.pallas_call(
        _final_body,
        out_shape=jax.ShapeDtypeStruct((N_NODES, OUT_DIM), jnp.float32),
    )(agg2a, agg2b, h2s, dinv_b, b2.reshape(1, -1),
      g2.reshape(1, -1), bt2.reshape(1, -1))

    return out


# speedup vs baseline: 2.5857x; 2.5857x over previous
import functools

import jax
import jax.numpy as jnp
from jax import lax
from jax.experimental import pallas as pl
from jax.experimental.pallas import tpu as pltpu
from jax.experimental.pallas import tpu_sc as plsc

N_NODES = 10000
N_EDGES = 320000
IN_DIM = 128
HID_DIM = 64
OUT_DIM = 128

NC = 2
NS = 16
NW = NC * NS
EPW = N_EDGES // NW
B = 125
NB = EPW // B
NBUF = 4
NPAD = 10240
RPS = NPAD // NS
DEG_W = 8

@functools.lru_cache(maxsize=None)
def _mesh():
    return plsc.VectorSubcoreMesh(
        core_axis_name="c", subcore_axis_name="s", num_cores=NC, num_subcores=NS
    )


def _deg_body(dst_hbm, ones_hbm, zeros_hbm, out_hbm,
              didx, ones_v, ssem, acc_sh, psem):
    c = lax.axis_index("c")
    s = lax.axis_index("s")
    wid = c * NS + s
    zdesc = pltpu.make_async_copy(
        zeros_hbm.at[pl.ds(s * RPS, RPS)], acc_sh.at[pl.ds(s * RPS, RPS)], psem
    )
    zdesc.start()
    pltpu.sync_copy(dst_hbm.at[wid], didx)
    pltpu.sync_copy(ones_hbm, ones_v)
    zdesc.wait()
    plsc.subcore_barrier()

    def scat(b, k):
        return pltpu.make_async_copy(ones_v, acc_sh.at[didx.at[b]], ssem[k])

    for k in range(NBUF):
        scat(k, k).start(add=True)

    def body(g, carry):
        for k in range(NBUF):
            b = g * NBUF + k
            scat(b, k).wait()
            scat(b + NBUF, k).start(add=True)
        return carry

    lax.fori_loop(0, NB // NBUF - 1, body, 0)
    for k in range(NBUF):
        scat(NB - NBUF + k, k).wait()
    plsc.subcore_barrier()
    pltpu.sync_copy(
        acc_sh.at[pl.ds(s * RPS, RPS)], out_hbm.at[c, pl.ds(s * RPS, RPS)]
    )


@functools.lru_cache(maxsize=None)
def _deg_kernel():
    return pl.kernel(
        _deg_body,
        out_type=jax.ShapeDtypeStruct((NC, NPAD, DEG_W), jnp.float32),
        mesh=_mesh(),
        scratch_types=[
            pltpu.VMEM((NB, B), jnp.int32),
            pltpu.VMEM((B, DEG_W), jnp.float32),
            tuple(pltpu.SemaphoreType.DMA for _ in range(NBUF)),
            pltpu.VMEM_SHARED((NPAD, DEG_W), jnp.float32),
            pltpu.SemaphoreType.DMA,
        ],
        compiler_params=pltpu.CompilerParams(use_tc_tiling_on_sc=False),
    )


def _make_agg(feat):
    def _agg_body(h_hbm, src_hbm, dst_hbm, zeros_hbm, out_hbm,
                  sidx, didx, rows, gsem, ssem, acc_sh, psem):
        c = lax.axis_index("c")
        s = lax.axis_index("s")
        wid = c * NS + s
        zdesc = pltpu.make_async_copy(
            zeros_hbm.at[pl.ds(s * RPS, RPS)],
            acc_sh.at[pl.ds(s * RPS, RPS)], psem
        )
        zdesc.start()
        pltpu.sync_copy(src_hbm.at[wid], sidx)
        pltpu.sync_copy(dst_hbm.at[wid], didx)
        zdesc.wait()
        plsc.subcore_barrier()

        def gat(b, k):
            return pltpu.make_async_copy(h_hbm.at[sidx.at[b]], rows[k], gsem[k])

        def scat(b, k):
            return pltpu.make_async_copy(rows[k], acc_sh.at[didx.at[b]], ssem[k])

        for k in range(NBUF):
            gat(k, k).start()

        def body(g, carry):
            for k in range(NBUF):
                b = g * NBUF + k
                gat(b, k).wait()
                scat(b, k).start(add=True)
            for k in range(NBUF):
                b = g * NBUF + k
                scat(b, k).wait()

                @pl.when(b + NBUF < NB)
                def _():
                    gat(b + NBUF, k).start()
            return carry

        lax.fori_loop(0, NB // NBUF, body, 0)
        plsc.subcore_barrier()
        pltpu.sync_copy(
            acc_sh.at[pl.ds(s * RPS, RPS)],
            out_hbm.at[c, pl.ds(s * RPS, RPS), pl.ds(0, feat)],
        )

    return pl.kernel(
        _agg_body,
        out_type=jax.ShapeDtypeStruct((NC, NPAD, 2 * feat), jnp.float32),
        mesh=_mesh(),
        scratch_types=[
            pltpu.VMEM((NB, B), jnp.int32),
            pltpu.VMEM((NB, B), jnp.int32),
            tuple(pltpu.VMEM((B, feat), jnp.float32) for _ in range(NBUF)),
            tuple(pltpu.SemaphoreType.DMA for _ in range(NBUF)),
            tuple(pltpu.SemaphoreType.DMA for _ in range(NBUF)),
            pltpu.VMEM_SHARED((NPAD, feat), jnp.float32),
            pltpu.SemaphoreType.DMA,
        ],
        compiler_params=pltpu.CompilerParams(use_tc_tiling_on_sc=False),
    )


_make_agg = functools.lru_cache(maxsize=None)(_make_agg)


def _mm_scale_body(x_ref, w_ref, dp_ref, o_ref, d_ref):
    dp = dp_ref[...]
    deg = dp[0, :N_NODES, :1] + dp[1, :N_NODES, :1] + 1.0
    dinv = lax.rsqrt(deg)
    h = jnp.dot(x_ref[...], w_ref[...], preferred_element_type=jnp.float32)
    h1s = h * dinv
    o_ref[...] = jnp.concatenate([h1s, jnp.zeros_like(h1s)], axis=1)
    d_ref[...] = jnp.broadcast_to(dinv, (N_NODES, OUT_DIM))


def _mid_body(agg_ref, hs_ref, dv_ref, b_ref, g_ref, bt_ref, w_ref, o_ref):
    dinv = dv_ref[:, :1]
    agg = agg_ref[...]
    acc = (agg[0, :N_NODES, :HID_DIM] + agg[1, :N_NODES, :HID_DIM]
           + hs_ref[:, :HID_DIM])
    pre = acc * dinv + b_ref[...]
    mu = jnp.mean(pre, axis=0, keepdims=True)
    var = jnp.mean((pre - mu) ** 2, axis=0, keepdims=True)
    bn = (pre - mu) * lax.rsqrt(var + 1e-5) * g_ref[...] + bt_ref[...]
    r = jnp.maximum(bn, 0.0)
    h2 = jnp.dot(r, w_ref[...], preferred_element_type=jnp.float32)
    o_ref[...] = h2 * dinv


def _final_body(agga_ref, aggb_ref, hs_ref, dv_ref,
                b_ref, g_ref, bt_ref, o_ref):
    dinv = dv_ref[:, :1]
    acc = jnp.concatenate(
        [agga_ref[0, :N_NODES, :HID_DIM] + agga_ref[1, :N_NODES, :HID_DIM],
         aggb_ref[0, :N_NODES, :HID_DIM] + aggb_ref[1, :N_NODES, :HID_DIM]],
        axis=1,
    ) + hs_ref[...]
    pre = acc * dinv + b_ref[...]
    mu = jnp.mean(pre, axis=0, keepdims=True)
    var = jnp.mean((pre - mu) ** 2, axis=0, keepdims=True)
    o_ref[...] = (pre - mu) * lax.rsqrt(var + 1e-5) * g_ref[...] + bt_ref[...]


def kernel(x, edge_index, W1, b1, g1, bt1, W2, b2, g2, bt2):
    dst = edge_index[1].reshape(NW, NB, B)
    src_a = (2 * edge_index[0]).reshape(NW, NB, B)
    src_b = src_a + 1
    ones8 = jnp.ones((B, DEG_W), jnp.float32)
    zeros8 = jnp.zeros((NPAD, DEG_W), jnp.float32)
    zeros64 = jnp.zeros((NPAD, HID_DIM), jnp.float32)

    deg_parts = _deg_kernel()(dst, ones8, zeros8)

    h1s, dinv_b = pl.pallas_call(
        _mm_scale_body,
        out_shape=(jax.ShapeDtypeStruct((N_NODES, OUT_DIM), jnp.float32),
                   jax.ShapeDtypeStruct((N_NODES, OUT_DIM), jnp.float32)),
    )(x, W1, deg_parts)

    agg = _make_agg(HID_DIM)
    agg1 = agg(h1s.reshape(2 * N_NODES, HID_DIM), src_a, dst, zeros64)

    h2s = pl.pallas_call(
        _mid_body,
        out_shape=jax.ShapeDtypeStruct((N_NODES, OUT_DIM), jnp.float32),
    )(agg1, h1s, dinv_b, b1.reshape(1, -1), g1.reshape(1, -1),
      bt1.reshape(1, -1), W2)

    h2v = h2s.reshape(2 * N_NODES, HID_DIM)
    agg2a = agg(h2v, src_a, dst, zeros64)
    agg2b = agg(h2v, src_b, dst, zeros64)

    out = pl.pallas_call(
        _final_body,
        out_shape=jax.ShapeDtypeStruct((N_NODES, OUT_DIM), jnp.float32),
    )(agg2a, agg2b, h2s, dinv_b, b2.reshape(1, -1),
      g2.reshape(1, -1), bt2.reshape(1, -1))

    return out


# speedup vs baseline: 2.7047x; 1.0460x over previous
import functools

import jax
import jax.numpy as jnp
from jax import lax
from jax.experimental import pallas as pl
from jax.experimental.pallas import tpu as pltpu
from jax.experimental.pallas import tpu_sc as plsc

N_NODES = 10000
N_EDGES = 320000
IN_DIM = 128
HID_DIM = 64
OUT_DIM = 128

NC = 2
NS = 16
NW = NC * NS
EPW = N_EDGES // NW
B = 125
NB = EPW // B
NBUF = 8
NPAD = 10240
RPS = NPAD // NS
DEG_W = 8

@functools.lru_cache(maxsize=None)
def _mesh():
    return plsc.VectorSubcoreMesh(
        core_axis_name="c", subcore_axis_name="s", num_cores=NC, num_subcores=NS
    )


def _deg_body(dst_hbm, ones_hbm, zeros_hbm, out_hbm,
              didx, ones_v, ssem, acc_sh, psem):
    c = lax.axis_index("c")
    s = lax.axis_index("s")
    wid = c * NS + s
    zdesc = pltpu.make_async_copy(
        zeros_hbm.at[pl.ds(s * RPS, RPS)], acc_sh.at[pl.ds(s * RPS, RPS)], psem
    )
    zdesc.start()
    pltpu.sync_copy(dst_hbm.at[wid], didx)
    pltpu.sync_copy(ones_hbm, ones_v)
    zdesc.wait()
    plsc.subcore_barrier()

    def scat(b, k):
        return pltpu.make_async_copy(ones_v, acc_sh.at[didx.at[b]], ssem[k])

    for k in range(NBUF):
        scat(k, k).start(add=True)

    def body(g, carry):
        for k in range(NBUF):
            b = g * NBUF + k
            scat(b, k).wait()
            scat(b + NBUF, k).start(add=True)
        return carry

    lax.fori_loop(0, NB // NBUF - 1, body, 0)
    for k in range(NBUF):
        scat(NB - NBUF + k, k).wait()
    plsc.subcore_barrier()
    pltpu.sync_copy(
        acc_sh.at[pl.ds(s * RPS, RPS)], out_hbm.at[c, pl.ds(s * RPS, RPS)]
    )


@functools.lru_cache(maxsize=None)
def _deg_kernel():
    return pl.kernel(
        _deg_body,
        out_type=jax.ShapeDtypeStruct((NC, NPAD, DEG_W), jnp.float32),
        mesh=_mesh(),
        scratch_types=[
            pltpu.VMEM((NB, B), jnp.int32),
            pltpu.VMEM((B, DEG_W), jnp.float32),
            tuple(pltpu.SemaphoreType.DMA for _ in range(NBUF)),
            pltpu.VMEM_SHARED((NPAD, DEG_W), jnp.float32),
            pltpu.SemaphoreType.DMA,
        ],
        compiler_params=pltpu.CompilerParams(use_tc_tiling_on_sc=False),
    )


def _make_agg(feat):
    def _agg_body(h_hbm, src_hbm, dst_hbm, zeros_hbm, out_hbm,
                  sidx, didx, rows, gsem, ssem, acc_sh, psem):
        c = lax.axis_index("c")
        s = lax.axis_index("s")
        wid = c * NS + s
        zdesc = pltpu.make_async_copy(
            zeros_hbm.at[pl.ds(s * RPS, RPS)],
            acc_sh.at[pl.ds(s * RPS, RPS)], psem
        )
        zdesc.start()
        pltpu.sync_copy(src_hbm.at[wid], sidx)
        pltpu.sync_copy(dst_hbm.at[wid], didx)
        zdesc.wait()
        plsc.subcore_barrier()

        def gat(b, k):
            return pltpu.make_async_copy(h_hbm.at[sidx.at[b]], rows[k], gsem[k])

        def scat(b, k):
            return pltpu.make_async_copy(rows[k], acc_sh.at[didx.at[b]], ssem[k])

        for k in range(NBUF):
            gat(k, k).start()

        def body(g, carry):
            for k in range(NBUF):
                b = g * NBUF + k
                gat(b, k).wait()
                scat(b, k).start(add=True)
            for k in range(NBUF):
                b = g * NBUF + k
                scat(b, k).wait()

                @pl.when(b + NBUF < NB)
                def _():
                    gat(b + NBUF, k).start()
            return carry

        lax.fori_loop(0, NB // NBUF, body, 0)
        plsc.subcore_barrier()
        pltpu.sync_copy(
            acc_sh.at[pl.ds(s * RPS, RPS)],
            out_hbm.at[c, pl.ds(s * RPS, RPS), pl.ds(0, feat)],
        )

    return pl.kernel(
        _agg_body,
        out_type=jax.ShapeDtypeStruct((NC, NPAD, 2 * feat), jnp.float32),
        mesh=_mesh(),
        scratch_types=[
            pltpu.VMEM((NB, B), jnp.int32),
            pltpu.VMEM((NB, B), jnp.int32),
            tuple(pltpu.VMEM((B, feat), jnp.float32) for _ in range(NBUF)),
            tuple(pltpu.SemaphoreType.DMA for _ in range(NBUF)),
            tuple(pltpu.SemaphoreType.DMA for _ in range(NBUF)),
            pltpu.VMEM_SHARED((NPAD, feat), jnp.float32),
            pltpu.SemaphoreType.DMA,
        ],
        compiler_params=pltpu.CompilerParams(use_tc_tiling_on_sc=False),
    )


_make_agg = functools.lru_cache(maxsize=None)(_make_agg)


def _mm_scale_body(x_ref, w_ref, dp_ref, o_ref, d_ref):
    dp = dp_ref[...]
    deg = dp[0, :N_NODES, :1] + dp[1, :N_NODES, :1] + 1.0
    dinv = lax.rsqrt(deg)
    h = jnp.dot(x_ref[...], w_ref[...], preferred_element_type=jnp.float32)
    h1s = h * dinv
    o_ref[...] = jnp.concatenate([h1s, jnp.zeros_like(h1s)], axis=1)
    d_ref[...] = jnp.broadcast_to(dinv, (N_NODES, OUT_DIM))


def _mid_body(agg_ref, hs_ref, dv_ref, b_ref, g_ref, bt_ref, w_ref, o_ref):
    dinv = dv_ref[:, :1]
    agg = agg_ref[...]
    acc = (agg[0, :N_NODES, :HID_DIM] + agg[1, :N_NODES, :HID_DIM]
           + hs_ref[:, :HID_DIM])
    pre = acc * dinv + b_ref[...]
    mu = jnp.mean(pre, axis=0, keepdims=True)
    var = jnp.mean((pre - mu) ** 2, axis=0, keepdims=True)
    bn = (pre - mu) * lax.rsqrt(var + 1e-5) * g_ref[...] + bt_ref[...]
    r = jnp.maximum(bn, 0.0)
    h2 = jnp.dot(r, w_ref[...], preferred_element_type=jnp.float32)
    o_ref[...] = h2 * dinv


def _final_body(agga_ref, aggb_ref, hs_ref, dv_ref,
                b_ref, g_ref, bt_ref, o_ref):
    dinv = dv_ref[:, :1]
    acc = jnp.concatenate(
        [agga_ref[0, :N_NODES, :HID_DIM] + agga_ref[1, :N_NODES, :HID_DIM],
         aggb_ref[0, :N_NODES, :HID_DIM] + aggb_ref[1, :N_NODES, :HID_DIM]],
        axis=1,
    ) + hs_ref[...]
    pre = acc * dinv + b_ref[...]
    mu = jnp.mean(pre, axis=0, keepdims=True)
    var = jnp.mean((pre - mu) ** 2, axis=0, keepdims=True)
    o_ref[...] = (pre - mu) * lax.rsqrt(var + 1e-5) * g_ref[...] + bt_ref[...]


def kernel(x, edge_index, W1, b1, g1, bt1, W2, b2, g2, bt2):
    dst = edge_index[1].reshape(NW, NB, B)
    src_a = (2 * edge_index[0]).reshape(NW, NB, B)
    src_b = src_a + 1
    ones8 = jnp.ones((B, DEG_W), jnp.float32)
    zeros8 = jnp.zeros((NPAD, DEG_W), jnp.float32)
    zeros64 = jnp.zeros((NPAD, HID_DIM), jnp.float32)

    deg_parts = _deg_kernel()(dst, ones8, zeros8)

    h1s, dinv_b = pl.pallas_call(
        _mm_scale_body,
        out_shape=(jax.ShapeDtypeStruct((N_NODES, OUT_DIM), jnp.float32),
                   jax.ShapeDtypeStruct((N_NODES, OUT_DIM), jnp.float32)),
    )(x, W1, deg_parts)

    agg = _make_agg(HID_DIM)
    agg1 = agg(h1s.reshape(2 * N_NODES, HID_DIM), src_a, dst, zeros64)

    h2s = pl.pallas_call(
        _mid_body,
        out_shape=jax.ShapeDtypeStruct((N_NODES, OUT_DIM), jnp.float32),
    )(agg1, h1s, dinv_b, b1.reshape(1, -1), g1.reshape(1, -1),
      bt1.reshape(1, -1), W2)

    h2v = h2s.reshape(2 * N_NODES, HID_DIM)
    agg2a = agg(h2v, src_a, dst, zeros64)
    agg2b = agg(h2v, src_b, dst, zeros64)

    out = pl.pallas_call(
        _final_body,
        out_shape=jax.ShapeDtypeStruct((N_NODES, OUT_DIM), jnp.float32),
    )(agg2a, agg2b, h2s, dinv_b, b2.reshape(1, -1),
      g2.reshape(1, -1), bt2.reshape(1, -1))

    return out
